# baseline (speedup 1.0000x reference)
import functools

import jax
import jax.numpy as jnp
import numpy as np
from jax import lax
from jax.experimental import pallas as pl
from jax.experimental.pallas import tpu as pltpu
from jax.experimental.pallas import tpu_sc as plsc

_VB = 1024
_K = 64
_CAP = 512


def _mm_flash_body(V):
    def body(hs_ref, w_ref, temp_ref, key_ref, m_ref, z_ref):
        i = pl.program_id(0)
        ls = jax.lax.dot_general(
            hs_ref[...], w_ref[...], (((1,), (1,)), ((), ())),
            preferred_element_type=jnp.float32)
        ls = ls / temp_ref[...]
        col = i * _VB + jax.lax.broadcasted_iota(jnp.int32, ls.shape, 1)
        ls = jnp.where(col < V, ls, -jnp.inf)
        bu = jax.lax.bitcast_convert_type(ls, jnp.uint32)
        key_ref[...] = jnp.where(ls < 0, ~bu, bu | jnp.uint32(0x80000000))
        bm = jnp.max(ls, axis=1, keepdims=True)
        bz = jnp.sum(jnp.exp(ls - bm), axis=1, keepdims=True)

        @pl.when(i == 0)
        def _():
            m_ref[...] = bm
            z_ref[...] = bz

        @pl.when(i > 0)
        def _():
            m_old = m_ref[...]
            z_old = z_ref[...]
            m_new = jnp.maximum(m_old, bm)
            z_ref[...] = (z_old * jnp.exp(m_old - m_new)
                          + bz * jnp.exp(bm - m_new))
            m_ref[...] = m_new

    return body


def _scaled_logits(embedder_weight, hs, temperatures):
    V, D = embedder_weight.shape
    B = hs.shape[0]
    grid = (pl.cdiv(V, _VB),)
    return pl.pallas_call(
        _mm_flash_body(V),
        grid=grid,
        in_specs=[
            pl.BlockSpec((B, D), lambda i: (0, 0)),
            pl.BlockSpec((_VB, D), lambda i: (i, 0)),
            pl.BlockSpec((B, 1), lambda i: (0, 0)),
        ],
        out_specs=[
            pl.BlockSpec((B, _VB), lambda i: (0, i)),
            pl.BlockSpec((B, 1), lambda i: (0, 0)),
            pl.BlockSpec((B, 1), lambda i: (0, 0)),
        ],
        out_shape=[
            jax.ShapeDtypeStruct((B, V), jnp.uint32),
            jax.ShapeDtypeStruct((B, 1), jnp.float32),
            jax.ShapeDtypeStruct((B, 1), jnp.float32),
        ],
        compiler_params=pltpu.CompilerParams(
            dimension_semantics=("arbitrary",)),
    )(hs, embedder_weight, temperatures.reshape(B, 1))


_NEGINF_KEY = 0x007FFFFF


def _sc_collect_topk(keys):
    B, V = keys.shape
    NB = V // 16
    NW = 32
    RP = B // NW
    mesh = plsc.VectorSubcoreMesh(core_axis_name="c", subcore_axis_name="s")

    @functools.partial(
        pl.kernel,
        mesh=mesh,
        out_type=[jax.ShapeDtypeStruct((B, _CAP), jnp.int32),
                  jax.ShapeDtypeStruct((B, _CAP), jnp.int32)],
        scratch_types=[
            pltpu.VMEM((V,), jnp.uint32),
            pltpu.VMEM((4096,), jnp.int32),
            pltpu.VMEM((_CAP,), jnp.int32),
            pltpu.VMEM((_CAP,), jnp.int32),
        ],
        compiler_params=pltpu.CompilerParams(needs_layout_passes=False),
    )
    def sc_topk(keys_hbm, ovals_hbm, oidx_hbm, buf, hist, cvals, cidx):
        lane = lax.iota(jnp.int32, 16)
        lane_u = lane.astype(jnp.uint32)
        ones = jnp.ones((16,), jnp.int32)
        zeros16 = jnp.zeros((16,), jnp.int32)
        wid = lax.axis_index("s") * 2 + lax.axis_index("c")

        UR = 10

        def zero_hist():
            def zb(j, c):
                for u in range(8):
                    hist[pl.ds((j * 8 + u) * 16, 16)] = zeros16
                return c
            lax.fori_loop(0, 32, zb, 0)

        def scan_hist(target):
            def sb(t, carry):
                found, b_sel, c_above, cum = carry
                b = 255 - t
                v = hist[pl.ds(b * 16, 16)]
                s = jnp.sum(v)
                cum2 = cum + s
                crossed = jnp.logical_and(found == 0, cum2 >= target)
                nb = jnp.where(crossed, b, b_sel)
                nc = jnp.where(crossed, cum, c_above)
                nf = jnp.where(crossed, 1, found)
                return (nf, nb, nc, cum2)
            out = lax.fori_loop(0, 256, sb, (0, 0, 0, 0))
            return out[1], out[2]

        def hist_pass(level, b1):
            b1u = lax.convert_element_type(b1, jnp.uint32)

            def body(jo, c):
                for u in range(UR):
                    key = buf[pl.ds((jo * UR + u) * 16, 16)]
                    if level == 1:
                        slot = (((key >> jnp.uint32(24)) << jnp.uint32(4))
                                | lane_u).astype(jnp.int32)
                        plsc.addupdate_scatter(hist, [slot], ones)
                    else:
                        inb = (key >> jnp.uint32(24)) == b1u
                        slot = ((((key >> jnp.uint32(16)) & jnp.uint32(0xFF))
                                 << jnp.uint32(4))
                                | lane_u).astype(jnp.int32)
                        plsc.addupdate_scatter(hist, [slot], ones, mask=inb)
                return c
            lax.fori_loop(0, NB // UR, body, 0)

        def collect(T):
            def body(jo, off):
                for u in range(UR):
                    j = jo * UR + u
                    key = buf[pl.ds(j * 16, 16)]
                    m = key >= T
                    mi = jnp.where(m, ones, zeros16)
                    pc = plsc.all_reduce_population_count(m)
                    ranks = off + (plsc.cumsum(mi) - mi)
                    ranks = jnp.minimum(ranks, _CAP - 1)
                    plsc.store_scatter(cvals, [ranks],
                                       key.astype(jnp.int32), mask=m)
                    plsc.store_scatter(cidx, [ranks], j * 16 + lane, mask=m)
                    off = off + pc
                return off
            lax.fori_loop(0, NB // UR, body, jnp.zeros((16,), jnp.int32))

        neginf = jnp.full((16,), _NEGINF_KEY, jnp.int32)

        for r in range(RP):
            row = wid * RP + r
            pltpu.sync_copy(keys_hbm.at[row], buf)

            def ib(j, c):
                cvals[pl.ds(j * 16, 16)] = neginf
                cidx[pl.ds(j * 16, 16)] = zeros16
                return c
            lax.fori_loop(0, _CAP // 16, ib, 0)

            zero_hist()
            hist_pass(1, 0)
            b1, c1 = scan_hist(64)
            zero_hist()
            hist_pass(2, b1)
            b2, _ = scan_hist(64 - c1)
            T = ((lax.convert_element_type(b1, jnp.uint32) << jnp.uint32(24))
                 | (lax.convert_element_type(b2, jnp.uint32)
                    << jnp.uint32(16)))
            collect(T)
            pltpu.sync_copy(cvals, ovals_hbm.at[row])
            pltpu.sync_copy(cidx, oidx_hbm.at[row])

    return sc_topk(keys)


def _rotl(x, r):
    return (x << np.uint32(r)) | (x >> np.uint32(32 - r))


def _threefry2x32(k0, k1, x0, x1):
    rot0 = (13, 15, 26, 6)
    rot1 = (17, 29, 16, 24)
    ks = [k0, k1, k0 ^ k1 ^ np.uint32(0x1BD11BDA)]
    x0 = x0 + ks[0]
    x1 = x1 + ks[1]
    for i in range(5):
        for r in (rot0 if i % 2 == 0 else rot1):
            x0 = x0 + x1
            x1 = _rotl(x1, r)
            x1 = x1 ^ x0
        x0 = x0 + ks[(i + 1) % 3]
        x1 = x1 + ks[(i + 2) % 3] + np.uint32(i + 1)
    return x0, x1


def _gumbel_at(flat_idx, seed):
    idx = flat_idx.astype(jnp.uint32)
    o0, o1 = _threefry2x32(jnp.uint32(0), jnp.uint32(seed),
                           jnp.zeros_like(idx), idx)
    bits = o0 ^ o1
    fb = (bits >> np.uint32(9)) | np.uint32(0x3F800000)
    u = jax.lax.bitcast_convert_type(fb, jnp.float32) - 1.0
    tiny = jnp.float32(np.finfo(np.float32).tiny)
    u = u * (jnp.float32(1.0) - tiny) + tiny
    u = jnp.maximum(tiny, u)
    return -jnp.log(-jnp.log(u))


def _sample_tail(vals, idxs, m, z, top_ps, top_ks, V):
    B, K = vals.shape
    p = jnp.exp(vals - m) / z
    cum = jnp.cumsum(p, axis=1)
    mask = ((cum - p) > top_ps[:, None]) | (
        jnp.arange(K)[None, :] >= top_ks[:, None])
    p_kept = jnp.where(mask, 0.0, p)
    p_final = p_kept / jnp.sum(p_kept, axis=1, keepdims=True)
    flat = (jnp.arange(B, dtype=jnp.uint32)[:, None] * jnp.uint32(V)
            + idxs.astype(jnp.uint32))
    g = _gumbel_at(flat, 42)
    score = jnp.log(p_final + 1e-30) + g
    win = jnp.argmax(score, axis=1)
    return jnp.take_along_axis(idxs, win[:, None], axis=1)[:, 0]


def kernel(embedder_weight, hidden_states, output_positions, temperatures,
           top_ps, top_ks):
    V, D = embedder_weight.shape
    B = hidden_states.shape[0]
    hs = jnp.take(hidden_states, output_positions, axis=1)[:, 0, :]
    keys, m, z = _scaled_logits(embedder_weight, hs, temperatures)
    cand_keys_i, cand_idx = _sc_collect_topk(keys)
    cand_keys = jax.lax.bitcast_convert_type(cand_keys_i, jnp.uint32)
    bits = jnp.where(cand_keys >= jnp.uint32(0x80000000),
                     cand_keys & jnp.uint32(0x7FFFFFFF), ~cand_keys)
    cand_vals = jax.lax.bitcast_convert_type(bits, jnp.float32)
    vals, pos = jax.lax.top_k(cand_vals, _K)
    idxs = jnp.take_along_axis(cand_idx, pos, axis=1)
    return _sample_tail(vals, idxs, m, z, top_ps, top_ks, V)


# speedup vs baseline: 1.9983x; 1.9983x over previous
import functools

import jax
import jax.numpy as jnp
import numpy as np
from jax import lax
from jax.experimental import pallas as pl
from jax.experimental.pallas import tpu as pltpu
from jax.experimental.pallas import tpu_sc as plsc

_VB = 1024
_K = 64
_CAP = 512


def _mm_flash_body(V):
    def body(hs_ref, w_ref, temp_ref, key_ref, m_ref, z_ref):
        i = pl.program_id(0)
        ls = jax.lax.dot_general(
            hs_ref[...], w_ref[...], (((1,), (1,)), ((), ())),
            preferred_element_type=jnp.float32)
        ls = ls / temp_ref[...]
        col = i * _VB + jax.lax.broadcasted_iota(jnp.int32, ls.shape, 1)
        ls = jnp.where(col < V, ls, -jnp.inf)
        bu = jax.lax.bitcast_convert_type(ls, jnp.uint32)
        key_ref[...] = jnp.where(ls < 0, ~bu, bu | jnp.uint32(0x80000000))
        bm = jnp.max(ls, axis=1, keepdims=True)
        bz = jnp.sum(jnp.exp(ls - bm), axis=1, keepdims=True)

        @pl.when(i == 0)
        def _():
            m_ref[...] = bm
            z_ref[...] = bz

        @pl.when(i > 0)
        def _():
            m_old = m_ref[...]
            z_old = z_ref[...]
            m_new = jnp.maximum(m_old, bm)
            z_ref[...] = (z_old * jnp.exp(m_old - m_new)
                          + bz * jnp.exp(bm - m_new))
            m_ref[...] = m_new

    return body


def _scaled_logits(embedder_weight, hs, temperatures):
    V, D = embedder_weight.shape
    B = hs.shape[0]
    grid = (pl.cdiv(V, _VB),)
    return pl.pallas_call(
        _mm_flash_body(V),
        grid=grid,
        in_specs=[
            pl.BlockSpec((B, D), lambda i: (0, 0)),
            pl.BlockSpec((_VB, D), lambda i: (i, 0)),
            pl.BlockSpec((B, 1), lambda i: (0, 0)),
        ],
        out_specs=[
            pl.BlockSpec((B, _VB), lambda i: (0, i)),
            pl.BlockSpec((B, 1), lambda i: (0, 0)),
            pl.BlockSpec((B, 1), lambda i: (0, 0)),
        ],
        out_shape=[
            jax.ShapeDtypeStruct((B, V), jnp.uint32),
            jax.ShapeDtypeStruct((B, 1), jnp.float32),
            jax.ShapeDtypeStruct((B, 1), jnp.float32),
        ],
        compiler_params=pltpu.CompilerParams(
            dimension_semantics=("arbitrary",)),
    )(hs, embedder_weight, temperatures.reshape(B, 1))


_NEGINF_KEY = 0x007FFFFF


def _sc_collect_topk(keys):
    B, V = keys.shape
    NB = V // 16
    NW = 32
    RP = B // NW
    mesh = plsc.VectorSubcoreMesh(core_axis_name="c", subcore_axis_name="s")

    @functools.partial(
        pl.kernel,
        mesh=mesh,
        out_type=[jax.ShapeDtypeStruct((B, _CAP), jnp.int32),
                  jax.ShapeDtypeStruct((B, _CAP), jnp.int32)],
        scratch_types=[
            pltpu.VMEM((V,), jnp.uint32),
            pltpu.VMEM((4096,), jnp.int32),
            pltpu.VMEM((_CAP,), jnp.int32),
            pltpu.VMEM((_CAP,), jnp.int32),
        ],
        compiler_params=pltpu.CompilerParams(needs_layout_passes=False),
    )
    def sc_topk(keys_hbm, ovals_hbm, oidx_hbm, buf, hist, cvals, cidx):
        lane = lax.iota(jnp.int32, 16)
        lane_u = lane.astype(jnp.uint32)
        ones = jnp.ones((16,), jnp.int32)
        zeros16 = jnp.zeros((16,), jnp.int32)
        wid = lax.axis_index("s") * 2 + lax.axis_index("c")

        UR = 10

        def zero_hist():
            def zb(j, c):
                for u in range(8):
                    hist[pl.ds((j * 8 + u) * 16, 16)] = zeros16
                return c
            lax.fori_loop(0, 32, zb, 0)

        def scan_hist(target):
            def sb(t, carry):
                found, b_sel, c_above, cum = carry
                b = 255 - t
                v = hist[pl.ds(b * 16, 16)]
                s = jnp.sum(v)
                cum2 = cum + s
                crossed = jnp.logical_and(found == 0, cum2 >= target)
                nb = jnp.where(crossed, b, b_sel)
                nc = jnp.where(crossed, cum, c_above)
                nf = jnp.where(crossed, 1, found)
                return (nf, nb, nc, cum2)
            out = lax.fori_loop(0, 256, sb, (0, 0, 0, 0))
            return out[1], out[2]

        def hist_pass(level, b1):
            b1u = lax.convert_element_type(b1, jnp.uint32)

            @plsc.parallel_loop(0, NB, unroll=UR)
            def body(j):
                key = buf[pl.ds(j * 16, 16)]
                if level == 1:
                    slot = (((key >> jnp.uint32(24)) << jnp.uint32(4))
                            | lane_u).astype(jnp.int32)
                    plsc.addupdate_scatter(hist, [slot], ones)
                else:
                    inb = (key >> jnp.uint32(24)) == b1u
                    slot = ((((key >> jnp.uint32(16)) & jnp.uint32(0xFF))
                             << jnp.uint32(4))
                            | lane_u).astype(jnp.int32)
                    plsc.addupdate_scatter(hist, [slot], ones, mask=inb)

        def collect(T):
            @plsc.parallel_loop(0, NB, unroll=UR,
                                carry=jnp.zeros((16,), jnp.int32))
            def body(j, off):
                key = buf[pl.ds(j * 16, 16)]
                m = key >= T
                mi = jnp.where(m, ones, zeros16)
                pc = plsc.all_reduce_population_count(m)
                ranks = off + (plsc.cumsum(mi) - mi)
                ranks = jnp.minimum(ranks, _CAP - 1)
                plsc.store_scatter(cvals, [ranks],
                                   key.astype(jnp.int32), mask=m)
                plsc.store_scatter(cidx, [ranks], j * 16 + lane, mask=m)
                return off + pc

        neginf = jnp.full((16,), _NEGINF_KEY, jnp.int32)

        for r in range(RP):
            row = wid * RP + r
            pltpu.sync_copy(keys_hbm.at[row], buf)

            def ib(j, c):
                cvals[pl.ds(j * 16, 16)] = neginf
                cidx[pl.ds(j * 16, 16)] = zeros16
                return c
            lax.fori_loop(0, _CAP // 16, ib, 0)

            zero_hist()
            hist_pass(1, 0)
            b1, c1 = scan_hist(64)
            zero_hist()
            hist_pass(2, b1)
            b2, _ = scan_hist(64 - c1)
            T = ((lax.convert_element_type(b1, jnp.uint32) << jnp.uint32(24))
                 | (lax.convert_element_type(b2, jnp.uint32)
                    << jnp.uint32(16)))
            collect(T)
            pltpu.sync_copy(cvals, ovals_hbm.at[row])
            pltpu.sync_copy(cidx, oidx_hbm.at[row])

    return sc_topk(keys)


def _rotl(x, r):
    return (x << np.uint32(r)) | (x >> np.uint32(32 - r))


def _threefry2x32(k0, k1, x0, x1):
    rot0 = (13, 15, 26, 6)
    rot1 = (17, 29, 16, 24)
    ks = [k0, k1, k0 ^ k1 ^ np.uint32(0x1BD11BDA)]
    x0 = x0 + ks[0]
    x1 = x1 + ks[1]
    for i in range(5):
        for r in (rot0 if i % 2 == 0 else rot1):
            x0 = x0 + x1
            x1 = _rotl(x1, r)
            x1 = x1 ^ x0
        x0 = x0 + ks[(i + 1) % 3]
        x1 = x1 + ks[(i + 2) % 3] + np.uint32(i + 1)
    return x0, x1


def _gumbel_at(flat_idx, seed):
    idx = flat_idx.astype(jnp.uint32)
    o0, o1 = _threefry2x32(jnp.uint32(0), jnp.uint32(seed),
                           jnp.zeros_like(idx), idx)
    bits = o0 ^ o1
    fb = (bits >> np.uint32(9)) | np.uint32(0x3F800000)
    u = jax.lax.bitcast_convert_type(fb, jnp.float32) - 1.0
    tiny = jnp.float32(np.finfo(np.float32).tiny)
    u = u * (jnp.float32(1.0) - tiny) + tiny
    u = jnp.maximum(tiny, u)
    return -jnp.log(-jnp.log(u))


def _sample_tail(vals, idxs, m, z, top_ps, top_ks, V):
    B, K = vals.shape
    p = jnp.exp(vals - m) / z
    cum = jnp.cumsum(p, axis=1)
    mask = ((cum - p) > top_ps[:, None]) | (
        jnp.arange(K)[None, :] >= top_ks[:, None])
    p_kept = jnp.where(mask, 0.0, p)
    p_final = p_kept / jnp.sum(p_kept, axis=1, keepdims=True)
    flat = (jnp.arange(B, dtype=jnp.uint32)[:, None] * jnp.uint32(V)
            + idxs.astype(jnp.uint32))
    g = _gumbel_at(flat, 42)
    score = jnp.log(p_final + 1e-30) + g
    win = jnp.argmax(score, axis=1)
    return jnp.take_along_axis(idxs, win[:, None], axis=1)[:, 0]


def kernel(embedder_weight, hidden_states, output_positions, temperatures,
           top_ps, top_ks):
    V, D = embedder_weight.shape
    B = hidden_states.shape[0]
    hs = jnp.take(hidden_states, output_positions, axis=1)[:, 0, :]
    keys, m, z = _scaled_logits(embedder_weight, hs, temperatures)
    cand_keys_i, cand_idx = _sc_collect_topk(keys)
    cand_keys = jax.lax.bitcast_convert_type(cand_keys_i, jnp.uint32)
    bits = jnp.where(cand_keys >= jnp.uint32(0x80000000),
                     cand_keys & jnp.uint32(0x7FFFFFFF), ~cand_keys)
    cand_vals = jax.lax.bitcast_convert_type(bits, jnp.float32)
    vals, pos = jax.lax.top_k(cand_vals, _K)
    idxs = jnp.take_along_axis(cand_idx, pos, axis=1)
    return _sample_tail(vals, idxs, m, z, top_ps, top_ks, V)


# speedup vs baseline: 2.2641x; 1.1330x over previous
import functools

import jax
import jax.numpy as jnp
import numpy as np
from jax import lax
from jax.experimental import pallas as pl
from jax.experimental.pallas import tpu as pltpu
from jax.experimental.pallas import tpu_sc as plsc

_VB = 2048
_K = 64
_CAP = 512


def _mm_flash_body(V):
    def body(hs_ref, w_ref, temp_ref, key_ref, m_ref, z_ref):
        i = pl.program_id(0)
        ls = jax.lax.dot_general(
            hs_ref[...], w_ref[...], (((1,), (1,)), ((), ())),
            preferred_element_type=jnp.float32)
        ls = ls / temp_ref[...]
        col = i * _VB + jax.lax.broadcasted_iota(jnp.int32, ls.shape, 1)
        ls = jnp.where(col < V, ls, -jnp.inf)
        bu = jax.lax.bitcast_convert_type(ls, jnp.uint32)
        key_ref[...] = jnp.where(ls < 0, ~bu, bu | jnp.uint32(0x80000000))
        bm = jnp.max(ls, axis=1, keepdims=True)
        bz = jnp.sum(jnp.exp(ls - bm), axis=1, keepdims=True)

        @pl.when(i == 0)
        def _():
            m_ref[...] = bm
            z_ref[...] = bz

        @pl.when(i > 0)
        def _():
            m_old = m_ref[...]
            z_old = z_ref[...]
            m_new = jnp.maximum(m_old, bm)
            z_ref[...] = (z_old * jnp.exp(m_old - m_new)
                          + bz * jnp.exp(bm - m_new))
            m_ref[...] = m_new

    return body


def _scaled_logits(embedder_weight, hs, temperatures):
    V, D = embedder_weight.shape
    B = hs.shape[0]
    grid = (pl.cdiv(V, _VB),)
    return pl.pallas_call(
        _mm_flash_body(V),
        grid=grid,
        in_specs=[
            pl.BlockSpec((B, D), lambda i: (0, 0)),
            pl.BlockSpec((_VB, D), lambda i: (i, 0)),
            pl.BlockSpec((B, 1), lambda i: (0, 0)),
        ],
        out_specs=[
            pl.BlockSpec((B, _VB), lambda i: (0, i)),
            pl.BlockSpec((B, 1), lambda i: (0, 0)),
            pl.BlockSpec((B, 1), lambda i: (0, 0)),
        ],
        out_shape=[
            jax.ShapeDtypeStruct((B, V), jnp.uint32),
            jax.ShapeDtypeStruct((B, 1), jnp.float32),
            jax.ShapeDtypeStruct((B, 1), jnp.float32),
        ],
        compiler_params=pltpu.CompilerParams(
            dimension_semantics=("arbitrary",)),
    )(hs, embedder_weight, temperatures.reshape(B, 1))


_NEGINF_KEY = 0x007FFFFF


def _sc_collect_topk(keys):
    B, V = keys.shape
    NB = V // 16
    NW = 32
    RP = B // NW
    mesh = plsc.VectorSubcoreMesh(core_axis_name="c", subcore_axis_name="s")

    @functools.partial(
        pl.kernel,
        mesh=mesh,
        out_type=[jax.ShapeDtypeStruct((B, _CAP), jnp.int32),
                  jax.ShapeDtypeStruct((B, _CAP), jnp.int32)],
        scratch_types=[
            pltpu.VMEM((V,), jnp.uint32),
            pltpu.VMEM((4096,), jnp.int32),
            pltpu.VMEM((_CAP,), jnp.int32),
            pltpu.VMEM((_CAP,), jnp.int32),
        ],
        compiler_params=pltpu.CompilerParams(needs_layout_passes=False),
    )
    def sc_topk(keys_hbm, ovals_hbm, oidx_hbm, buf, hist, cvals, cidx):
        lane = lax.iota(jnp.int32, 16)
        lane_u = lane.astype(jnp.uint32)
        ones = jnp.ones((16,), jnp.int32)
        zeros16 = jnp.zeros((16,), jnp.int32)
        wid = lax.axis_index("s") * 2 + lax.axis_index("c")

        UR = 10

        def zero_hist():
            def zb(j, c):
                for u in range(8):
                    hist[pl.ds((j * 8 + u) * 16, 16)] = zeros16
                return c
            lax.fori_loop(0, 32, zb, 0)

        def scan_hist(target):
            def sb(t, carry):
                found, b_sel, c_above, cum = carry
                b = 255 - t
                v = hist[pl.ds(b * 16, 16)]
                s = jnp.sum(v)
                cum2 = cum + s
                crossed = jnp.logical_and(found == 0, cum2 >= target)
                nb = jnp.where(crossed, b, b_sel)
                nc = jnp.where(crossed, cum, c_above)
                nf = jnp.where(crossed, 1, found)
                return (nf, nb, nc, cum2)
            out = lax.fori_loop(0, 256, sb, (0, 0, 0, 0))
            return out[1], out[2]

        def hist_pass(level, b1):
            b1u = lax.convert_element_type(b1, jnp.uint32)

            @plsc.parallel_loop(0, NB, unroll=UR)
            def body(j):
                key = buf[pl.ds(j * 16, 16)]
                if level == 1:
                    slot = (((key >> jnp.uint32(24)) << jnp.uint32(4))
                            | lane_u).astype(jnp.int32)
                    plsc.addupdate_scatter(hist, [slot], ones)
                else:
                    inb = (key >> jnp.uint32(24)) == b1u
                    slot = ((((key >> jnp.uint32(16)) & jnp.uint32(0xFF))
                             << jnp.uint32(4))
                            | lane_u).astype(jnp.int32)
                    plsc.addupdate_scatter(hist, [slot], ones, mask=inb)

        def collect(T):
            @plsc.parallel_loop(0, NB, unroll=UR,
                                carry=jnp.zeros((16,), jnp.int32))
            def body(j, off):
                key = buf[pl.ds(j * 16, 16)]
                m = key >= T
                mi = jnp.where(m, ones, zeros16)
                pc = plsc.all_reduce_population_count(m)
                ranks = off + (plsc.cumsum(mi) - mi)
                ranks = jnp.minimum(ranks, _CAP - 1)
                plsc.store_scatter(cvals, [ranks],
                                   key.astype(jnp.int32), mask=m)
                plsc.store_scatter(cidx, [ranks], j * 16 + lane, mask=m)
                return off + pc

        neginf = jnp.full((16,), _NEGINF_KEY, jnp.int32)

        for r in range(RP):
            row = wid * RP + r
            pltpu.sync_copy(keys_hbm.at[row], buf)

            def ib(j, c):
                cvals[pl.ds(j * 16, 16)] = neginf
                cidx[pl.ds(j * 16, 16)] = zeros16
                return c
            lax.fori_loop(0, _CAP // 16, ib, 0)

            zero_hist()
            hist_pass(1, 0)
            b1, c1 = scan_hist(64)
            zero_hist()
            hist_pass(2, b1)
            b2, _ = scan_hist(64 - c1)
            T = ((lax.convert_element_type(b1, jnp.uint32) << jnp.uint32(24))
                 | (lax.convert_element_type(b2, jnp.uint32)
                    << jnp.uint32(16)))
            collect(T)
            pltpu.sync_copy(cvals, ovals_hbm.at[row])
            pltpu.sync_copy(cidx, oidx_hbm.at[row])

    return sc_topk(keys)


def _rotl(x, r):
    return (x << np.uint32(r)) | (x >> np.uint32(32 - r))


def _threefry2x32(k0, k1, x0, x1):
    rot0 = (13, 15, 26, 6)
    rot1 = (17, 29, 16, 24)
    ks = [k0, k1, k0 ^ k1 ^ np.uint32(0x1BD11BDA)]
    x0 = x0 + ks[0]
    x1 = x1 + ks[1]
    for i in range(5):
        for r in (rot0 if i % 2 == 0 else rot1):
            x0 = x0 + x1
            x1 = _rotl(x1, r)
            x1 = x1 ^ x0
        x0 = x0 + ks[(i + 1) % 3]
        x1 = x1 + ks[(i + 2) % 3] + np.uint32(i + 1)
    return x0, x1


def _gumbel_at(flat_idx, seed):
    idx = flat_idx.astype(jnp.uint32)
    o0, o1 = _threefry2x32(jnp.uint32(0), jnp.uint32(seed),
                           jnp.zeros_like(idx), idx)
    bits = o0 ^ o1
    fb = (bits >> np.uint32(9)) | np.uint32(0x3F800000)
    u = jax.lax.bitcast_convert_type(fb, jnp.float32) - 1.0
    tiny = jnp.float32(np.finfo(np.float32).tiny)
    u = u * (jnp.float32(1.0) - tiny) + tiny
    u = jnp.maximum(tiny, u)
    return -jnp.log(-jnp.log(u))


def _sample_tail(vals, idxs, m, z, top_ps, top_ks, V):
    B, K = vals.shape
    p = jnp.exp(vals - m) / z
    cum = jnp.cumsum(p, axis=1)
    mask = ((cum - p) > top_ps[:, None]) | (
        jnp.arange(K)[None, :] >= top_ks[:, None])
    p_kept = jnp.where(mask, 0.0, p)
    p_final = p_kept / jnp.sum(p_kept, axis=1, keepdims=True)
    flat = (jnp.arange(B, dtype=jnp.uint32)[:, None] * jnp.uint32(V)
            + idxs.astype(jnp.uint32))
    g = _gumbel_at(flat, 42)
    score = jnp.log(p_final + 1e-30) + g
    win = jnp.argmax(score, axis=1)
    return jnp.take_along_axis(idxs, win[:, None], axis=1)[:, 0]


def kernel(embedder_weight, hidden_states, output_positions, temperatures,
           top_ps, top_ks):
    V, D = embedder_weight.shape
    B = hidden_states.shape[0]
    hs = jnp.take(hidden_states, output_positions, axis=1)[:, 0, :]
    keys, m, z = _scaled_logits(embedder_weight, hs, temperatures)
    cand_keys_i, cand_idx = _sc_collect_topk(keys)
    cand_keys = jax.lax.bitcast_convert_type(cand_keys_i, jnp.uint32)
    bits = jnp.where(cand_keys >= jnp.uint32(0x80000000),
                     cand_keys & jnp.uint32(0x7FFFFFFF), ~cand_keys)
    cand_vals = jax.lax.bitcast_convert_type(bits, jnp.float32)
    vals, pos = jax.lax.top_k(cand_vals, _K)
    idxs = jnp.take_along_axis(cand_idx, pos, axis=1)
    return _sample_tail(vals, idxs, m, z, top_ps, top_ks, V)


# speedup vs baseline: 2.3057x; 1.0184x over previous
import functools

import jax
import jax.numpy as jnp
import numpy as np
from jax import lax
from jax.experimental import pallas as pl
from jax.experimental.pallas import tpu as pltpu
from jax.experimental.pallas import tpu_sc as plsc

_VB = 4096
_K = 64
_CAP = 512


def _mm_flash_body(V):
    def body(hs_ref, w_ref, temp_ref, key_ref, m_ref, z_ref):
        i = pl.program_id(0)
        ls = jax.lax.dot_general(
            hs_ref[...], w_ref[...], (((1,), (1,)), ((), ())),
            preferred_element_type=jnp.float32)
        ls = ls / temp_ref[...]
        col = i * _VB + jax.lax.broadcasted_iota(jnp.int32, ls.shape, 1)
        ls = jnp.where(col < V, ls, -jnp.inf)
        bu = jax.lax.bitcast_convert_type(ls, jnp.uint32)
        key_ref[...] = jnp.where(ls < 0, ~bu, bu | jnp.uint32(0x80000000))
        bm = jnp.max(ls, axis=1, keepdims=True)
        bz = jnp.sum(jnp.exp(ls - bm), axis=1, keepdims=True)

        @pl.when(i == 0)
        def _():
            m_ref[...] = bm
            z_ref[...] = bz

        @pl.when(i > 0)
        def _():
            m_old = m_ref[...]
            z_old = z_ref[...]
            m_new = jnp.maximum(m_old, bm)
            z_ref[...] = (z_old * jnp.exp(m_old - m_new)
                          + bz * jnp.exp(bm - m_new))
            m_ref[...] = m_new

    return body


def _scaled_logits(embedder_weight, hs, temperatures):
    V, D = embedder_weight.shape
    B = hs.shape[0]
    grid = (pl.cdiv(V, _VB),)
    return pl.pallas_call(
        _mm_flash_body(V),
        grid=grid,
        in_specs=[
            pl.BlockSpec((B, D), lambda i: (0, 0)),
            pl.BlockSpec((_VB, D), lambda i: (i, 0)),
            pl.BlockSpec((B, 1), lambda i: (0, 0)),
        ],
        out_specs=[
            pl.BlockSpec((B, _VB), lambda i: (0, i)),
            pl.BlockSpec((B, 1), lambda i: (0, 0)),
            pl.BlockSpec((B, 1), lambda i: (0, 0)),
        ],
        out_shape=[
            jax.ShapeDtypeStruct((B, V), jnp.uint32),
            jax.ShapeDtypeStruct((B, 1), jnp.float32),
            jax.ShapeDtypeStruct((B, 1), jnp.float32),
        ],
        compiler_params=pltpu.CompilerParams(
            dimension_semantics=("arbitrary",)),
    )(hs, embedder_weight, temperatures.reshape(B, 1))


_NEGINF_KEY = 0x007FFFFF


def _sc_collect_topk(keys):
    B, V = keys.shape
    NB = V // 16
    NW = 32
    RP = B // NW
    mesh = plsc.VectorSubcoreMesh(core_axis_name="c", subcore_axis_name="s")

    @functools.partial(
        pl.kernel,
        mesh=mesh,
        out_type=[jax.ShapeDtypeStruct((B, _CAP), jnp.int32),
                  jax.ShapeDtypeStruct((B, _CAP), jnp.int32)],
        scratch_types=[
            pltpu.VMEM((V,), jnp.uint32),
            pltpu.VMEM((4096,), jnp.int32),
            pltpu.VMEM((_CAP,), jnp.int32),
            pltpu.VMEM((_CAP,), jnp.int32),
        ],
        compiler_params=pltpu.CompilerParams(needs_layout_passes=False),
    )
    def sc_topk(keys_hbm, ovals_hbm, oidx_hbm, buf, hist, cvals, cidx):
        lane = lax.iota(jnp.int32, 16)
        lane_u = lane.astype(jnp.uint32)
        ones = jnp.ones((16,), jnp.int32)
        zeros16 = jnp.zeros((16,), jnp.int32)
        wid = lax.axis_index("s") * 2 + lax.axis_index("c")

        UR = 10

        def zero_hist():
            def zb(j, c):
                for u in range(8):
                    hist[pl.ds((j * 8 + u) * 16, 16)] = zeros16
                return c
            lax.fori_loop(0, 32, zb, 0)

        def scan_hist(target):
            def sb(t, carry):
                found, b_sel, c_above, cum = carry
                b = 255 - t
                v = hist[pl.ds(b * 16, 16)]
                s = jnp.sum(v)
                cum2 = cum + s
                crossed = jnp.logical_and(found == 0, cum2 >= target)
                nb = jnp.where(crossed, b, b_sel)
                nc = jnp.where(crossed, cum, c_above)
                nf = jnp.where(crossed, 1, found)
                return (nf, nb, nc, cum2)
            out = lax.fori_loop(0, 256, sb, (0, 0, 0, 0))
            return out[1], out[2]

        def hist_pass(level, b1):
            b1u = lax.convert_element_type(b1, jnp.uint32)

            @plsc.parallel_loop(0, NB, unroll=UR)
            def body(j):
                key = buf[pl.ds(j * 16, 16)]
                if level == 1:
                    slot = (((key >> jnp.uint32(24)) << jnp.uint32(4))
                            | lane_u).astype(jnp.int32)
                    plsc.addupdate_scatter(hist, [slot], ones)
                else:
                    inb = (key >> jnp.uint32(24)) == b1u
                    slot = ((((key >> jnp.uint32(16)) & jnp.uint32(0xFF))
                             << jnp.uint32(4))
                            | lane_u).astype(jnp.int32)
                    plsc.addupdate_scatter(hist, [slot], ones, mask=inb)

        def collect(T):
            @plsc.parallel_loop(0, NB, unroll=UR,
                                carry=jnp.zeros((16,), jnp.int32))
            def body(j, off):
                key = buf[pl.ds(j * 16, 16)]
                m = key >= T
                mi = jnp.where(m, ones, zeros16)
                pc = plsc.all_reduce_population_count(m)
                ranks = off + (plsc.cumsum(mi) - mi)
                ranks = jnp.minimum(ranks, _CAP - 1)
                plsc.store_scatter(cvals, [ranks],
                                   key.astype(jnp.int32), mask=m)
                plsc.store_scatter(cidx, [ranks], j * 16 + lane, mask=m)
                return off + pc

        neginf = jnp.full((16,), _NEGINF_KEY, jnp.int32)

        for r in range(RP):
            row = wid * RP + r
            pltpu.sync_copy(keys_hbm.at[row], buf)

            def ib(j, c):
                cvals[pl.ds(j * 16, 16)] = neginf
                cidx[pl.ds(j * 16, 16)] = zeros16
                return c
            lax.fori_loop(0, _CAP // 16, ib, 0)

            zero_hist()
            hist_pass(1, 0)
            b1, c1 = scan_hist(64)
            zero_hist()
            hist_pass(2, b1)
            b2, _ = scan_hist(64 - c1)
            T = ((lax.convert_element_type(b1, jnp.uint32) << jnp.uint32(24))
                 | (lax.convert_element_type(b2, jnp.uint32)
                    << jnp.uint32(16)))
            collect(T)
            pltpu.sync_copy(cvals, ovals_hbm.at[row])
            pltpu.sync_copy(cidx, oidx_hbm.at[row])

    return sc_topk(keys)


def _rotl(x, r):
    return (x << np.uint32(r)) | (x >> np.uint32(32 - r))


def _threefry2x32(k0, k1, x0, x1):
    rot0 = (13, 15, 26, 6)
    rot1 = (17, 29, 16, 24)
    ks = [k0, k1, k0 ^ k1 ^ np.uint32(0x1BD11BDA)]
    x0 = x0 + ks[0]
    x1 = x1 + ks[1]
    for i in range(5):
        for r in (rot0 if i % 2 == 0 else rot1):
            x0 = x0 + x1
            x1 = _rotl(x1, r)
            x1 = x1 ^ x0
        x0 = x0 + ks[(i + 1) % 3]
        x1 = x1 + ks[(i + 2) % 3] + np.uint32(i + 1)
    return x0, x1


def _gumbel_at(flat_idx, seed):
    idx = flat_idx.astype(jnp.uint32)
    o0, o1 = _threefry2x32(jnp.uint32(0), jnp.uint32(seed),
                           jnp.zeros_like(idx), idx)
    bits = o0 ^ o1
    fb = (bits >> np.uint32(9)) | np.uint32(0x3F800000)
    u = jax.lax.bitcast_convert_type(fb, jnp.float32) - 1.0
    tiny = jnp.float32(np.finfo(np.float32).tiny)
    u = u * (jnp.float32(1.0) - tiny) + tiny
    u = jnp.maximum(tiny, u)
    return -jnp.log(-jnp.log(u))


def _sample_tail(vals, idxs, m, z, top_ps, top_ks, V):
    B, K = vals.shape
    p = jnp.exp(vals - m) / z
    cum = jnp.cumsum(p, axis=1)
    mask = ((cum - p) > top_ps[:, None]) | (
        jnp.arange(K)[None, :] >= top_ks[:, None])
    p_kept = jnp.where(mask, 0.0, p)
    p_final = p_kept / jnp.sum(p_kept, axis=1, keepdims=True)
    flat = (jnp.arange(B, dtype=jnp.uint32)[:, None] * jnp.uint32(V)
            + idxs.astype(jnp.uint32))
    g = _gumbel_at(flat, 42)
    score = jnp.log(p_final + 1e-30) + g
    win = jnp.argmax(score, axis=1)
    return jnp.take_along_axis(idxs, win[:, None], axis=1)[:, 0]


def kernel(embedder_weight, hidden_states, output_positions, temperatures,
           top_ps, top_ks):
    V, D = embedder_weight.shape
    B = hidden_states.shape[0]
    hs = jnp.take(hidden_states, output_positions, axis=1)[:, 0, :]
    keys, m, z = _scaled_logits(embedder_weight, hs, temperatures)
    cand_keys_i, cand_idx = _sc_collect_topk(keys)
    cand_keys = jax.lax.bitcast_convert_type(cand_keys_i, jnp.uint32)
    bits = jnp.where(cand_keys >= jnp.uint32(0x80000000),
                     cand_keys & jnp.uint32(0x7FFFFFFF), ~cand_keys)
    cand_vals = jax.lax.bitcast_convert_type(bits, jnp.float32)
    vals, pos = jax.lax.top_k(cand_vals, _K)
    idxs = jnp.take_along_axis(cand_idx, pos, axis=1)
    return _sample_tail(vals, idxs, m, z, top_ps, top_ks, V)
